# baseline (speedup 1.0000x reference)
import functools

import jax
import jax.numpy as jnp
from jax import lax
from jax.experimental import pallas as pl
from jax.experimental.pallas import tpu as pltpu
from jax.experimental.pallas import tpu_sc as plsc

N = 10000
D = 128
NC = 2
NS = 16
L = 16
DH = D // NC
CHUNK = 100
NBUF = 4


def _matmul_kernel(x_ref, w_ref, b_ref, h_ref):
    h_ref[0] = (
        lax.dot_general(x_ref[...], w_ref[0], (((1,), (1,)), ((), ())),
                        preferred_element_type=jnp.float32)
        + b_ref[0]
    )


def _scale_rows(buf, w_v, j):
    done = 0
    while done < CHUNK:
        lo = min(done, CHUNK - L)
        wrow = w_v[j, pl.ds(lo, L)]
        for u in range(done - lo, L):
            b = lo + u
            wsplat = wrow[jnp.full((L,), u, jnp.int32)]
            for d in range(DH // L):
                sl = pl.ds(d * L, L)
                buf[b, sl] = buf[b, sl] * wsplat
        done = lo + L


def _spmm_body(n_chunks, h_hbm, col_hbm, row_hbm, w_hbm, zeros_hbm, out_hbm,
               col_v, row_v, w_v, buf0, buf1, buf2, buf3,
               acc_smem, gsem0, gsem1, gsem2, gsem3,
               ssem0, ssem1, ssem2, ssem3):
    c = lax.axis_index("c")
    s = lax.axis_index("s")
    bufs = (buf0, buf1, buf2, buf3)
    gsems = (gsem0, gsem1, gsem2, gsem3)
    ssems = (ssem0, ssem1, ssem2, ssem3)

    pltpu.sync_copy(col_hbm.at[s], col_v)
    pltpu.sync_copy(row_hbm.at[s], row_v)
    pltpu.sync_copy(w_hbm.at[s], w_v)

    n_row_chunks = N // 80
    pltpu.sync_copy(zeros_hbm, buf0.at[pl.ds(0, 80)])
    for i in range(-(-n_row_chunks // NS)):
        k = s + NS * i

        @pl.when(k < n_row_chunks)
        def _():
            pltpu.sync_copy(buf0.at[pl.ds(0, 80)],
                            acc_smem.at[pl.ds(k * 80, 80)])
    plsc.subcore_barrier()

    hc = h_hbm.at[c]
    pltpu.async_copy(hc.at[col_v.at[0]], buf0, gsem0)
    pltpu.async_copy(hc.at[col_v.at[1]], buf1, gsem1)

    def step(j, k):
        buf, gsem, ssem = bufs[k], gsems[k], ssems[k]
        kp = (k - 2) % NBUF
        pltpu.make_async_copy(hc.at[col_v.at[j]], buf, gsem).wait()
        _scale_rows(buf, w_v, j)
        pltpu.async_copy(buf, acc_smem.at[row_v.at[j]], ssem, add=True)

        @pl.when(j >= 2)
        def _():
            pltpu.make_async_copy(bufs[kp], acc_smem.at[row_v.at[j - 2]],
                                  ssems[kp]).wait()

        @pl.when(j + 2 < n_chunks)
        def _():
            pltpu.async_copy(hc.at[col_v.at[j + 2]], bufs[kp], gsems[kp])

    def group_body(g, _):
        j = NBUF * g
        for k in range(NBUF):
            step(j + k, k)
        return 0

    lax.fori_loop(0, n_chunks // NBUF, group_body, 0)

    pltpu.make_async_copy(buf2, acc_smem.at[row_v.at[n_chunks - 2]],
                          ssem2).wait()
    pltpu.make_async_copy(buf3, acc_smem.at[row_v.at[n_chunks - 1]],
                          ssem3).wait()
    plsc.subcore_barrier()

    for i in range(-(-n_row_chunks // NS)):
        k = s + NS * i

        @pl.when(k < n_row_chunks)
        def _():
            pltpu.sync_copy(acc_smem.at[pl.ds(k * 80, 80)],
                            buf0.at[pl.ds(0, 80)])
            pltpu.sync_copy(buf0.at[pl.ds(0, 80)],
                            out_hbm.at[pl.ds(k * 80, 80), pl.ds(c * DH, DH)])


def _spmm(h_split, col, row, w, n_chunks):
    mesh = plsc.VectorSubcoreMesh(
        core_axis_name="c", subcore_axis_name="s", num_cores=NC, num_subcores=NS)
    zeros = jnp.zeros((80, DH), jnp.float32)
    kern = pl.kernel(
        functools.partial(_spmm_body, n_chunks),
        out_type=jax.ShapeDtypeStruct((N, D), jnp.float32),
        mesh=mesh,
        compiler_params=pltpu.CompilerParams(use_tc_tiling_on_sc=False),
        scratch_types=[
            pltpu.VMEM((n_chunks, CHUNK), jnp.int32),
            pltpu.VMEM((n_chunks, CHUNK), jnp.int32),
            pltpu.VMEM((n_chunks, CHUNK), jnp.float32),
            pltpu.VMEM((CHUNK, DH), jnp.float32),
            pltpu.VMEM((CHUNK, DH), jnp.float32),
            pltpu.VMEM((CHUNK, DH), jnp.float32),
            pltpu.VMEM((CHUNK, DH), jnp.float32),
            pltpu.VMEM_SHARED((N, DH), jnp.float32),
            pltpu.SemaphoreType.DMA,
            pltpu.SemaphoreType.DMA,
            pltpu.SemaphoreType.DMA,
            pltpu.SemaphoreType.DMA,
            pltpu.SemaphoreType.DMA,
            pltpu.SemaphoreType.DMA,
            pltpu.SemaphoreType.DMA,
            pltpu.SemaphoreType.DMA,
        ],
    )
    return kern(h_split, col, row, w, zeros)


def kernel(x, edge_index, edge_weight, W, b):
    E = edge_index.shape[1]
    per_tile = -(-E // NS)
    n_chunks = -(-per_tile // CHUNK)
    n_chunks += (-n_chunks) % NBUF
    e_pad = NS * n_chunks * CHUNK

    pad = e_pad - E
    if pad:
        pad_idx = (jnp.arange(pad, dtype=jnp.int32) * 7) % N
        col = jnp.concatenate([edge_index[1], pad_idx])
        row = jnp.concatenate([edge_index[0], pad_idx])
        w = jnp.concatenate([edge_weight, jnp.zeros((pad,), jnp.float32)])
    else:
        col, row, w = edge_index[1], edge_index[0], edge_weight
    col = col.reshape(NS, n_chunks, CHUNK)
    row = row.reshape(NS, n_chunks, CHUNK)
    w = w.reshape(NS, n_chunks, CHUNK)

    n_rows = x.shape[0]
    blk = 1000
    h_split = pl.pallas_call(
        _matmul_kernel,
        grid=(NC, n_rows // blk),
        in_specs=[
            pl.BlockSpec((blk, D), lambda c, i: (i, 0)),
            pl.BlockSpec((1, DH, D), lambda c, i: (c, 0, 0)),
            pl.BlockSpec((1, 1, DH), lambda c, i: (c, 0, 0)),
        ],
        out_specs=pl.BlockSpec((1, blk, DH), lambda c, i: (c, i, 0)),
        out_shape=jax.ShapeDtypeStruct((NC, n_rows, DH), jnp.float32),
    )(x, W.reshape(NC, DH, D), b.reshape(NC, 1, DH))

    return _spmm(h_split, col, row, w, n_chunks)


# speedup vs baseline: 1.0731x; 1.0731x over previous
import functools

import jax
import jax.numpy as jnp
from jax import lax
from jax.experimental import pallas as pl
from jax.experimental.pallas import tpu as pltpu
from jax.experimental.pallas import tpu_sc as plsc

N = 10000
D = 128
NC = 2
NS = 16
L = 16
DH = D // NC
CHUNK = 100
NBUF = 4


def _matmul_kernel(x_ref, w_ref, b_ref, h_ref):
    h_ref[0] = (
        lax.dot_general(x_ref[...], w_ref[...], (((1,), (1,)), ((), ())),
                        preferred_element_type=jnp.float32)
        + b_ref[0]
    )


def _scale_rows(buf, w_v, j):
    done = 0
    while done < CHUNK:
        lo = min(done, CHUNK - L)
        wrow = w_v[j, pl.ds(lo, L)]
        for u in range(done - lo, L):
            b = lo + u
            wsplat = wrow[jnp.full((L,), u, jnp.int32)]
            for d in range(DH // L):
                sl = pl.ds(d * L, L)
                buf[b, sl] = buf[b, sl] * wsplat
        done = lo + L


def _spmm_body(n_chunks, h_hbm, ei_hbm, w_hbm, zeros_hbm, out_hbm,
               col_v, row_v, w_v, buf0, buf1, buf2, buf3,
               acc_smem, gsem0, gsem1, gsem2, gsem3,
               ssem0, ssem1, ssem2, ssem3):
    c = lax.axis_index("c")
    s = lax.axis_index("s")
    bufs = (buf0, buf1, buf2, buf3)
    gsems = (gsem0, gsem1, gsem2, gsem3)
    ssems = (ssem0, ssem1, ssem2, ssem3)

    pltpu.sync_copy(ei_hbm.at[1].at[s], col_v)
    pltpu.sync_copy(ei_hbm.at[0].at[s], row_v)
    pltpu.sync_copy(w_hbm.at[s], w_v)

    n_row_chunks = N // 80
    pltpu.sync_copy(zeros_hbm, buf0.at[pl.ds(0, 80)])
    for i in range(-(-n_row_chunks // NS)):
        k = s + NS * i

        @pl.when(k < n_row_chunks)
        def _():
            pltpu.sync_copy(buf0.at[pl.ds(0, 80)],
                            acc_smem.at[pl.ds(k * 80, 80)])
    plsc.subcore_barrier()

    hc = h_hbm.at[c]
    pltpu.async_copy(hc.at[col_v.at[0]], buf0, gsem0)
    pltpu.async_copy(hc.at[col_v.at[1]], buf1, gsem1)

    def step(j, k):
        buf, gsem, ssem = bufs[k], gsems[k], ssems[k]
        kp = (k - 2) % NBUF
        pltpu.make_async_copy(hc.at[col_v.at[j]], buf, gsem).wait()
        _scale_rows(buf, w_v, j)
        pltpu.async_copy(buf, acc_smem.at[row_v.at[j]], ssem, add=True)

        @pl.when(j >= 2)
        def _():
            pltpu.make_async_copy(bufs[kp], acc_smem.at[row_v.at[j - 2]],
                                  ssems[kp]).wait()

        @pl.when(j + 2 < n_chunks)
        def _():
            pltpu.async_copy(hc.at[col_v.at[j + 2]], bufs[kp], gsems[kp])

    def group_body(g, _):
        j = NBUF * g
        for k in range(NBUF):
            step(j + k, k)
        return 0

    lax.fori_loop(0, n_chunks // NBUF, group_body, 0)

    pltpu.make_async_copy(buf2, acc_smem.at[row_v.at[n_chunks - 2]],
                          ssem2).wait()
    pltpu.make_async_copy(buf3, acc_smem.at[row_v.at[n_chunks - 1]],
                          ssem3).wait()
    plsc.subcore_barrier()

    for i in range(-(-n_row_chunks // NS)):
        k = s + NS * i

        @pl.when(k < n_row_chunks)
        def _():
            pltpu.sync_copy(acc_smem.at[pl.ds(k * 80, 80)],
                            buf0.at[pl.ds(0, 80)])
            pltpu.sync_copy(buf0.at[pl.ds(0, 80)],
                            out_hbm.at[pl.ds(k * 80, 80), pl.ds(c * DH, DH)])


def _spmm(h_split, ei, w, n_chunks):
    mesh = plsc.VectorSubcoreMesh(
        core_axis_name="c", subcore_axis_name="s", num_cores=NC, num_subcores=NS)
    zeros = jnp.zeros((80, DH), jnp.float32)
    kern = pl.kernel(
        functools.partial(_spmm_body, n_chunks),
        out_type=jax.ShapeDtypeStruct((N, D), jnp.float32),
        mesh=mesh,
        compiler_params=pltpu.CompilerParams(use_tc_tiling_on_sc=False),
        scratch_types=[
            pltpu.VMEM((n_chunks, CHUNK), jnp.int32),
            pltpu.VMEM((n_chunks, CHUNK), jnp.int32),
            pltpu.VMEM((n_chunks, CHUNK), jnp.float32),
            pltpu.VMEM((CHUNK, DH), jnp.float32),
            pltpu.VMEM((CHUNK, DH), jnp.float32),
            pltpu.VMEM((CHUNK, DH), jnp.float32),
            pltpu.VMEM((CHUNK, DH), jnp.float32),
            pltpu.VMEM_SHARED((N, DH), jnp.float32),
            pltpu.SemaphoreType.DMA,
            pltpu.SemaphoreType.DMA,
            pltpu.SemaphoreType.DMA,
            pltpu.SemaphoreType.DMA,
            pltpu.SemaphoreType.DMA,
            pltpu.SemaphoreType.DMA,
            pltpu.SemaphoreType.DMA,
            pltpu.SemaphoreType.DMA,
        ],
    )
    return kern(h_split, ei, w, zeros)


def kernel(x, edge_index, edge_weight, W, b):
    E = edge_index.shape[1]
    per_tile = -(-E // NS)
    n_chunks = -(-per_tile // CHUNK)
    n_chunks += (-n_chunks) % NBUF
    e_pad = NS * n_chunks * CHUNK

    pad = e_pad - E
    if pad:
        pad_idx = (jnp.arange(pad, dtype=jnp.int32) * 7) % N
        pad2 = jnp.stack([pad_idx, pad_idx])
        ei = jnp.concatenate([edge_index, pad2], axis=1)
        w = jnp.concatenate([edge_weight, jnp.zeros((pad,), jnp.float32)])
    else:
        ei, w = edge_index, edge_weight
    ei = ei.reshape(2, NS, n_chunks, CHUNK)
    w = w.reshape(NS, n_chunks, CHUNK)

    n_rows = x.shape[0]
    blk = 5000
    h_split = pl.pallas_call(
        _matmul_kernel,
        grid=(NC, n_rows // blk),
        in_specs=[
            pl.BlockSpec((blk, D), lambda c, i: (i, 0)),
            pl.BlockSpec((DH, D), lambda c, i: (c, 0)),
            pl.BlockSpec((1, 1, DH), lambda c, i: (c, 0, 0)),
        ],
        out_specs=pl.BlockSpec((1, blk, DH), lambda c, i: (c, i, 0)),
        out_shape=jax.ShapeDtypeStruct((NC, n_rows, DH), jnp.float32),
    )(x, W, b.reshape(NC, 1, DH))

    return _spmm(h_split, ei, w, n_chunks)


# speedup vs baseline: 1.0997x; 1.0248x over previous
import functools

import jax
import jax.numpy as jnp
from jax import lax
from jax.experimental import pallas as pl
from jax.experimental.pallas import tpu as pltpu
from jax.experimental.pallas import tpu_sc as plsc

N = 10000
D = 128
NC = 2
NS = 16
L = 16
DH = D // NC
CHUNK = 100
NBUF = 4


def _matmul_kernel(x_ref, w_ref, b_ref, h_ref):
    x = x_ref[...]
    for c in range(NC):
        h_ref[c] = (
            lax.dot_general(x, w_ref[pl.ds(c * DH, DH), :],
                            (((1,), (1,)), ((), ())),
                            preferred_element_type=jnp.float32)
            + b_ref[c]
        )


def _scale_rows(buf, w_v, j):
    done = 0
    while done < CHUNK:
        lo = min(done, CHUNK - L)
        wrow = w_v[j, pl.ds(lo, L)]
        for u in range(done - lo, L):
            b = lo + u
            wsplat = wrow[jnp.full((L,), u, jnp.int32)]
            for d in range(DH // L):
                sl = pl.ds(d * L, L)
                buf[b, sl] = buf[b, sl] * wsplat
        done = lo + L


def _spmm_body(n_chunks, h_hbm, ei_hbm, w_hbm, zeros_hbm, out_hbm,
               col_v, row_v, w_v, buf0, buf1, buf2, buf3,
               acc_smem, gsem0, gsem1, gsem2, gsem3,
               ssem0, ssem1, ssem2, ssem3):
    c = lax.axis_index("c")
    s = lax.axis_index("s")
    bufs = (buf0, buf1, buf2, buf3)
    gsems = (gsem0, gsem1, gsem2, gsem3)
    ssems = (ssem0, ssem1, ssem2, ssem3)

    pltpu.async_copy(ei_hbm.at[1].at[s], col_v, gsem0)
    pltpu.async_copy(ei_hbm.at[0].at[s], row_v, gsem1)
    pltpu.async_copy(w_hbm.at[s], w_v, gsem2)

    n_row_chunks = N // 80
    pltpu.sync_copy(zeros_hbm, buf0.at[pl.ds(0, 80)])
    for i in range(-(-n_row_chunks // NS)):
        k = s + NS * i

        @pl.when(k < n_row_chunks)
        def _():
            pltpu.sync_copy(buf0.at[pl.ds(0, 80)],
                            acc_smem.at[pl.ds(k * 80, 80)])
    pltpu.make_async_copy(ei_hbm.at[1].at[s], col_v, gsem0).wait()
    pltpu.make_async_copy(ei_hbm.at[0].at[s], row_v, gsem1).wait()
    pltpu.make_async_copy(w_hbm.at[s], w_v, gsem2).wait()
    plsc.subcore_barrier()

    hc = h_hbm.at[c]
    pltpu.async_copy(hc.at[col_v.at[0]], buf0, gsem0)
    pltpu.async_copy(hc.at[col_v.at[1]], buf1, gsem1)

    def step(j, k):
        buf, gsem, ssem = bufs[k], gsems[k], ssems[k]
        kp = (k - 2) % NBUF
        pltpu.make_async_copy(hc.at[col_v.at[j]], buf, gsem).wait()
        _scale_rows(buf, w_v, j)
        pltpu.async_copy(buf, acc_smem.at[row_v.at[j]], ssem, add=True)

        @pl.when(j >= 2)
        def _():
            pltpu.make_async_copy(bufs[kp], acc_smem.at[row_v.at[j - 2]],
                                  ssems[kp]).wait()

        @pl.when(j + 2 < n_chunks)
        def _():
            pltpu.async_copy(hc.at[col_v.at[j + 2]], bufs[kp], gsems[kp])

    def group_body(g, _):
        j = NBUF * g
        for k in range(NBUF):
            step(j + k, k)
        return 0

    lax.fori_loop(0, n_chunks // NBUF, group_body, 0)

    pltpu.make_async_copy(buf2, acc_smem.at[row_v.at[n_chunks - 2]],
                          ssem2).wait()
    pltpu.make_async_copy(buf3, acc_smem.at[row_v.at[n_chunks - 1]],
                          ssem3).wait()
    plsc.subcore_barrier()

    for i in range(-(-n_row_chunks // NS)):
        k = s + NS * i

        @pl.when(k < n_row_chunks)
        def _():
            pltpu.sync_copy(acc_smem.at[pl.ds(k * 80, 80)],
                            buf0.at[pl.ds(0, 80)])
            pltpu.sync_copy(buf0.at[pl.ds(0, 80)],
                            out_hbm.at[pl.ds(k * 80, 80), pl.ds(c * DH, DH)])


def _spmm(h_split, ei, w, n_chunks):
    mesh = plsc.VectorSubcoreMesh(
        core_axis_name="c", subcore_axis_name="s", num_cores=NC, num_subcores=NS)
    zeros = jnp.zeros((80, DH), jnp.float32)
    kern = pl.kernel(
        functools.partial(_spmm_body, n_chunks),
        out_type=jax.ShapeDtypeStruct((N, D), jnp.float32),
        mesh=mesh,
        compiler_params=pltpu.CompilerParams(use_tc_tiling_on_sc=False),
        scratch_types=[
            pltpu.VMEM((n_chunks, CHUNK), jnp.int32),
            pltpu.VMEM((n_chunks, CHUNK), jnp.int32),
            pltpu.VMEM((n_chunks, CHUNK), jnp.float32),
            pltpu.VMEM((CHUNK, DH), jnp.float32),
            pltpu.VMEM((CHUNK, DH), jnp.float32),
            pltpu.VMEM((CHUNK, DH), jnp.float32),
            pltpu.VMEM((CHUNK, DH), jnp.float32),
            pltpu.VMEM_SHARED((N, DH), jnp.float32),
            pltpu.SemaphoreType.DMA,
            pltpu.SemaphoreType.DMA,
            pltpu.SemaphoreType.DMA,
            pltpu.SemaphoreType.DMA,
            pltpu.SemaphoreType.DMA,
            pltpu.SemaphoreType.DMA,
            pltpu.SemaphoreType.DMA,
            pltpu.SemaphoreType.DMA,
        ],
    )
    return kern(h_split, ei, w, zeros)


def kernel(x, edge_index, edge_weight, W, b):
    E = edge_index.shape[1]
    per_tile = -(-E // NS)
    n_chunks = -(-per_tile // CHUNK)
    n_chunks += (-n_chunks) % NBUF
    e_pad = NS * n_chunks * CHUNK

    pad = e_pad - E
    if pad:
        pad_idx = (jnp.arange(pad, dtype=jnp.int32) * 7) % N
        pad2 = jnp.stack([pad_idx, pad_idx])
        ei = jnp.concatenate([edge_index, pad2], axis=1)
        w = jnp.concatenate([edge_weight, jnp.zeros((pad,), jnp.float32)])
    else:
        ei, w = edge_index, edge_weight
    ei = ei.reshape(2, NS, n_chunks, CHUNK)
    w = w.reshape(NS, n_chunks, CHUNK)

    n_rows = x.shape[0]
    blk = 5000
    h_split = pl.pallas_call(
        _matmul_kernel,
        grid=(n_rows // blk,),
        in_specs=[
            pl.BlockSpec((blk, D), lambda i: (i, 0)),
            pl.BlockSpec((D, D), lambda i: (0, 0)),
            pl.BlockSpec((NC, 1, DH), lambda i: (0, 0, 0)),
        ],
        out_specs=pl.BlockSpec((NC, blk, DH), lambda i: (0, i, 0)),
        out_shape=jax.ShapeDtypeStruct((NC, n_rows, DH), jnp.float32),
    )(x, W, b.reshape(NC, 1, DH))

    return _spmm(h_split, ei, w, n_chunks)
